# Initial kernel scaffold; baseline (speedup 1.0000x reference)
import jax
import jax.numpy as jnp
from jax.experimental import pallas as pl


def kernel(x, edge_index, batch, W_emb, b_emb, Wp, Vp, bp, Wq, Vq, bq, W1, b1, W2, b2):
    raise NotImplementedError("write your pallas kernel here")


# speedup vs baseline: 1.8023x; 1.8023x over previous
import functools

import jax
import jax.numpy as jnp
from jax import lax
from jax.experimental import pallas as pl
from jax.experimental.pallas import tpu as pltpu
from jax.experimental.pallas import tpu_sc as plsc

NUM_LAYERS = 10
EPSILON = 0.1

NC = 2
NS = 16
NT = NC * NS

_RPT = 313
_CAP = 10816
_K = 64
_NCH = _CAP // _K


def _route(sort_key, gather_idx, N):
    E = sort_key.shape[0]
    order = jnp.argsort(sort_key, stable=True)
    key_s = sort_key[order]
    gid_s = gather_idx[order]
    tile = jnp.minimum(key_s // _RPT, NT - 1)
    local = key_s - tile * _RPT
    eptr = jnp.searchsorted(key_s, jnp.arange(NT, dtype=sort_key.dtype) * _RPT)
    pos = jnp.arange(E, dtype=jnp.int32) - eptr[tile].astype(jnp.int32)
    pos_i = (pos % _NCH) * _K + pos // _NCH
    dest = tile.astype(jnp.int32) * _CAP + pos_i
    sub = tile // NC
    slot_val = sub.astype(jnp.int32) * (_RPT + 1) + local.astype(jnp.int32)
    gid = jnp.zeros((NT * _CAP,), jnp.int32).at[dest].set(gid_s.astype(jnp.int32))
    trash = (jnp.arange(NT * _CAP, dtype=jnp.int32) // _CAP // NC) * (_RPT + 1) + _RPT
    slot = trash.at[dest].set(slot_val)
    return gid, slot


@functools.lru_cache(maxsize=None)
def _make_spmm(N, D):
    NP = NT * _RPT
    mesh = plsc.VectorSubcoreMesh(core_axis_name="c", subcore_axis_name="s")

    @functools.partial(
        pl.kernel,
        out_type=jax.ShapeDtypeStruct((NP, D), jnp.float32),
        mesh=mesh,
        compiler_params=pltpu.CompilerParams(use_tc_tiling_on_sc=False),
        scratch_types=[
            pltpu.VMEM((_K,), jnp.int32),
            pltpu.VMEM((_K,), jnp.int32),
            pltpu.VMEM((_K, D), jnp.float32),
            pltpu.VMEM((_RPT + 1, D), jnp.float32),
            pltpu.VMEM_SHARED((NS * (_RPT + 1), D), jnp.float32),
            pltpu.SemaphoreType.DMA,
        ],
    )
    def spmm(y_hbm, gid_hbm, slot_hbm, zeros_hbm, out_hbm, gi_v, si_v,
             rows_v, stage_v, acc_s, sem):
        c = lax.axis_index("c")
        s = lax.axis_index("s")
        t = s * NC + c
        abase = s * (_RPT + 1)

        pltpu.sync_copy(zeros_hbm, stage_v)
        pltpu.sync_copy(stage_v, acc_s.at[pl.ds(abase, _RPT + 1)])

        ebase = t * _CAP

        def body(i, _):
            base = ebase + i * _K
            pltpu.sync_copy(gid_hbm.at[pl.ds(base, _K)], gi_v)
            pltpu.sync_copy(slot_hbm.at[pl.ds(base, _K)], si_v)
            pltpu.async_copy(y_hbm.at[gi_v], rows_v, sem).wait()
            pltpu.sync_copy(rows_v, acc_s.at[si_v], add=True)
            return 0

        lax.fori_loop(0, _NCH, body, 0)

        pltpu.sync_copy(acc_s.at[pl.ds(abase, _RPT)],
                        stage_v.at[pl.ds(0, _RPT)])
        pltpu.sync_copy(stage_v.at[pl.ds(0, _RPT)],
                        out_hbm.at[pl.ds(t * _RPT, _RPT)])

    return spmm


_DN = (((1,), (1,)), ((), ()))
_DN2 = (((1,), (0,)), ((), ()))


def _dense_a_body(y_ref, agg_ref, w_ref, v_ref, b_ref, o_ref):
    z = lax.dot_general(y_ref[...], w_ref[...], _DN,
                        preferred_element_type=jnp.float32)
    z += lax.dot_general(agg_ref[...], v_ref[...], _DN,
                         preferred_element_type=jnp.float32)
    o_ref[...] = jnp.tanh(z + b_ref[...])


def _make_dense_a(N, D, blk=2000):
    wspec = pl.BlockSpec((D, D), lambda i: (0, 0))
    return pl.pallas_call(
        _dense_a_body,
        grid=(N // blk,),
        in_specs=[
            pl.BlockSpec((blk, D), lambda i: (i, 0)),
            pl.BlockSpec((blk, D), lambda i: (i, 0)),
            wspec, wspec,
            pl.BlockSpec((1, D), lambda i: (0, 0)),
        ],
        out_specs=pl.BlockSpec((blk, D), lambda i: (i, 0)),
        out_shape=jax.ShapeDtypeStruct((N, D), jnp.float32),
    )


def _make_update(N, D, eps, blk=2000):
    def body(base_ref, a_ref, back_ref, w_ref, v_ref, o_ref):
        g = lax.dot_general(a_ref[...], w_ref[...], _DN2,
                            preferred_element_type=jnp.float32)
        g += lax.dot_general(back_ref[...], v_ref[...], _DN2,
                             preferred_element_type=jnp.float32)
        o_ref[...] = base_ref[...] + eps * g

    wspec = pl.BlockSpec((D, D), lambda i: (0, 0))
    return pl.pallas_call(
        body,
        grid=(N // blk,),
        in_specs=[
            pl.BlockSpec((blk, D), lambda i: (i, 0)),
            pl.BlockSpec((blk, D), lambda i: (i, 0)),
            pl.BlockSpec((blk, D), lambda i: (i, 0)),
            wspec, wspec,
        ],
        out_specs=pl.BlockSpec((blk, D), lambda i: (i, 0)),
        out_shape=jax.ShapeDtypeStruct((N, D), jnp.float32),
    )


def _make_embed(N, D_IN, D, blk=2000):
    def body(x_ref, w_ref, b_ref, o_ref):
        o_ref[...] = lax.dot_general(
            x_ref[...], w_ref[...], _DN,
            preferred_element_type=jnp.float32) + b_ref[...]

    return pl.pallas_call(
        body,
        grid=(N // blk,),
        in_specs=[
            pl.BlockSpec((blk, D_IN), lambda i: (i, 0)),
            pl.BlockSpec((D, D_IN), lambda i: (0, 0)),
            pl.BlockSpec((1, D), lambda i: (0, 0)),
        ],
        out_specs=pl.BlockSpec((blk, D), lambda i: (i, 0)),
        out_shape=jax.ShapeDtypeStruct((N, D), jnp.float32),
    )


def _leaky(x):
    return jnp.where(x >= 0, x, 0.01 * x)


def _make_readout(N, D, OUT, blk=2000):
    def body(p_ref, q_ref, w1a_ref, w1b_ref, b1_ref, w2_ref, b2_ref, o_ref):
        h = lax.dot_general(p_ref[...], w1a_ref[...], _DN,
                            preferred_element_type=jnp.float32)
        h += lax.dot_general(q_ref[...], w1b_ref[...], _DN,
                             preferred_element_type=jnp.float32)
        h = _leaky(h + b1_ref[...])
        o = lax.dot_general(h, w2_ref[...], _DN,
                            preferred_element_type=jnp.float32)
        o_ref[...] = _leaky(o + b2_ref[...])

    wspec = pl.BlockSpec((D, D), lambda i: (0, 0))
    bspec = pl.BlockSpec((1, D), lambda i: (0, 0))
    return pl.pallas_call(
        body,
        grid=(N // blk,),
        in_specs=[
            pl.BlockSpec((blk, D), lambda i: (i, 0)),
            pl.BlockSpec((blk, D), lambda i: (i, 0)),
            wspec, wspec, bspec,
            pl.BlockSpec((OUT, D), lambda i: (0, 0)),
            pl.BlockSpec((1, OUT), lambda i: (0, 0)),
        ],
        out_specs=pl.BlockSpec((blk, OUT), lambda i: (i, 0)),
        out_shape=jax.ShapeDtypeStruct((N, OUT), jnp.float32),
    )


def kernel(x, edge_index, batch, W_emb, b_emb, Wp, Vp, bp, Wq, Vq, bq,
           W1, b1, W2, b2):
    N, D_IN = x.shape
    D = W_emb.shape[0]
    OUT = W2.shape[0]
    src = edge_index[0].astype(jnp.int32)
    dst = edge_index[1].astype(jnp.int32)

    gid_f, slot_f = _route(dst, src, N)
    gid_b, slot_b = _route(src, dst, N)
    zeros_blk = jnp.zeros((_RPT + 1, D), jnp.float32)

    spmm0 = _make_spmm(N, D)

    def spmm(y, gid, slot):
        return spmm0(y, gid, slot, zeros_blk)[:N]

    dense_a = _make_dense_a(N, D)
    upd_p = _make_update(N, D, EPSILON)
    upd_q = _make_update(N, D, -EPSILON)
    embed = _make_embed(N, D_IN, D)
    readout = _make_readout(N, D, OUT)

    h = embed(x, W_emb, b_emb.reshape(1, D))
    p = h
    q = h
    bq2 = bq.reshape(1, D)
    bp2 = bp.reshape(1, D)
    for _ in range(NUM_LAYERS):
        agg = spmm(q, gid_f, slot_f)
        a = dense_a(q, agg, Wq, Vq, bq2)
        back = spmm(a, gid_b, slot_b)
        p = upd_p(p, a, back, Wq, Vq)

        agg = spmm(p, gid_f, slot_f)
        a = dense_a(p, agg, Wp, Vp, bp2)
        back = spmm(a, gid_b, slot_b)
        q = upd_q(q, a, back, Wp, Vp)

    return readout(p, q, W1[:, :D], W1[:, D:], b1.reshape(1, D),
                   W2, b2.reshape(1, OUT))
